# Initial kernel scaffold; baseline (speedup 1.0000x reference)
import jax
import jax.numpy as jnp
from jax.experimental import pallas as pl


def kernel(x, tok_table, pos_table):
    raise NotImplementedError("write your pallas kernel here")


# speedup vs baseline: 2.2446x; 2.2446x over previous
import functools

import jax
import jax.numpy as jnp
from jax import lax
from jax.experimental import pallas as pl
from jax.experimental.pallas import tpu as pltpu
from jax.experimental.pallas import tpu_sc as plsc

DIM = 64
LANES = 16
NSLICE = DIM // LANES


def kernel(x, tok_table, pos_table):
    B, S = x.shape
    xT = x.astype(jnp.int32).T

    NW = 32
    BW = B // NW

    mesh = plsc.VectorSubcoreMesh(core_axis_name="c", subcore_axis_name="s")

    @functools.partial(
        pl.kernel,
        mesh=mesh,
        out_type=jax.ShapeDtypeStruct((B, S, DIM), jnp.float32),
        compiler_params=pltpu.CompilerParams(use_tc_tiling_on_sc=False),
        scratch_types=[
            pltpu.VMEM((BW,), jnp.int32),
            pltpu.VMEM((BW, DIM), jnp.float32),
            pltpu.VMEM((S, DIM), jnp.float32),
            pltpu.SemaphoreType.DMA,
        ],
    )
    def sc_kernel(xT_hbm, tok_hbm, pos_hbm, out_hbm, idx_v, rows_v, pos_v, sem):
        wid = lax.axis_index("s") * 2 + lax.axis_index("c")
        b0 = wid * BW
        pltpu.sync_copy(pos_hbm.at[pl.ds(0, S)], pos_v)

        def p_body(p, carry):
            pltpu.sync_copy(xT_hbm.at[p, pl.ds(b0, BW)], idx_v)
            pltpu.async_copy(tok_hbm.at[idx_v], rows_v, sem).wait()
            pregs = [pos_v[p, pl.ds(d * LANES, LANES)] for d in range(NSLICE)]

            def r_body(r, c2):
                for d in range(NSLICE):
                    sl = pl.ds(d * LANES, LANES)
                    rows_v[r, sl] = rows_v[r, sl] + pregs[d]
                return c2

            lax.fori_loop(0, BW, r_body, 0)
            pltpu.sync_copy(rows_v, out_hbm.at[pl.ds(b0, BW), p])
            return carry

        lax.fori_loop(0, S, p_body, 0)

    return sc_kernel(xT, tok_table, pos_table)


# speedup vs baseline: 2.8605x; 1.2744x over previous
import functools

import jax
import jax.numpy as jnp
from jax import lax
from jax.experimental import pallas as pl
from jax.experimental.pallas import tpu as pltpu
from jax.experimental.pallas import tpu_sc as plsc

DIM = 64
LANES = 16
NSLICE = DIM // LANES
NBUF = 4
UNROLL = 4


def kernel(x, tok_table, pos_table):
    B, S = x.shape
    xT = x.astype(jnp.int32).T

    NW = 32
    BW = B // NW

    mesh = plsc.VectorSubcoreMesh(core_axis_name="c", subcore_axis_name="s")

    @functools.partial(
        pl.kernel,
        mesh=mesh,
        out_type=jax.ShapeDtypeStruct((B, S, DIM), jnp.float32),
        compiler_params=pltpu.CompilerParams(use_tc_tiling_on_sc=False),
        scratch_types=[
            pltpu.VMEM((S, BW), jnp.int32),
            pltpu.VMEM((NBUF, BW, DIM), jnp.float32),
            pltpu.VMEM((S, DIM), jnp.float32),
            [pltpu.SemaphoreType.DMA for _ in range(NBUF)],
            [pltpu.SemaphoreType.DMA for _ in range(NBUF)],
        ],
    )
    def sc_kernel(xT_hbm, tok_hbm, pos_hbm, out_hbm,
                  idx_all, rows_b, pos_v, gsems, ssems):
        wid = lax.axis_index("s") * 2 + lax.axis_index("c")
        b0 = wid * BW
        pltpu.sync_copy(pos_hbm.at[pl.ds(0, S)], pos_v)
        pltpu.sync_copy(xT_hbm.at[pl.ds(0, S), pl.ds(b0, BW)], idx_all)

        def gather_cp(p, b):
            return pltpu.make_async_copy(
                tok_hbm.at[idx_all.at[p]], rows_b.at[b], gsems[b])

        def scatter_cp(p, b):
            return pltpu.make_async_copy(
                rows_b.at[b], out_hbm.at[pl.ds(b0, BW), p], ssems[b])

        gather_cp(0, 0).start()
        gather_cp(1, 1).start()

        def outer(pp, carry):
            for u in range(NBUF):
                p = pp * NBUF + u
                b2 = (u + 2) % NBUF

                @pl.when(p >= 2)
                def _():
                    scatter_cp(p - 2, b2).wait()

                @pl.when(p + 2 < S)
                def _():
                    gather_cp(p + 2, b2).start()

                gather_cp(p, u).wait()
                pregs = [pos_v[p, pl.ds(d * LANES, LANES)]
                         for d in range(NSLICE)]

                def r_body(rq, c2):
                    r0 = rq * UNROLL
                    for v in range(UNROLL):
                        for d in range(NSLICE):
                            sl = pl.ds(d * LANES, LANES)
                            rows_b[u, r0 + v, sl] = (
                                rows_b[u, r0 + v, sl] + pregs[d])
                    return c2

                lax.fori_loop(0, BW // UNROLL, r_body, 0)
                scatter_cp(p, u).start()
            return carry

        lax.fori_loop(0, S // NBUF, outer, 0)
        scatter_cp(S - 2, 2).wait()
        scatter_cp(S - 1, 3).wait()

    return sc_kernel(xT, tok_table, pos_table)
